# Initial kernel scaffold; baseline (speedup 1.0000x reference)
import jax
import jax.numpy as jnp
from jax.experimental import pallas as pl


def kernel(pos, atomic_numbers, batch, edge_index, W_embed, W_lift, R1, R2, R3, W_self_s, W_self_v, W_sc_s, W_sc_v, W_out1, W_out2):
    raise NotImplementedError("write your pallas kernel here")


# speedup vs baseline: 12.8538x; 12.8538x over previous
import functools
import math

import jax
import jax.numpy as jnp
from jax import lax
from jax.experimental import pallas as pl
from jax.experimental.pallas import tpu as pltpu
from jax.experimental.pallas import tpu_sc as plsc

F32 = jnp.float32
I32 = jnp.int32

NCORES = 2
NSUB = 16
NW = NCORES * NSUB
LANES = 16
CHUNK = 128

R_MAX = 4.0
NUM_BASIS = 8
P_CUT = 6.0
N_CH = 16
INV_AVG = 1.0 / 16.0
SQRT3 = math.sqrt(3.0)
PREF = math.sqrt(2.0 / R_MAX)

BE = 4096
BN = 2000


def _ru(x, m):
    return (x + m - 1) // m * m


def _mesh():
    return plsc.VectorSubcoreMesh(
        core_axis_name="c", subcore_axis_name="s",
        num_cores=NCORES, num_subcores=NSUB)


_SC_PARAMS = pltpu.CompilerParams(use_tc_tiling_on_sc=False)


def _silu(x):
    return x * jax.nn.sigmoid(x)


def _sc_edge_vec(E_pad, NPOS):
    EW = E_pad // NW
    NCH = EW // CHUNK

    scratch = (
        [pltpu.VMEM((CHUNK,), I32) for _ in range(4)]
        + [pltpu.VMEM((CHUNK, 16), F32) for _ in range(4)]
        + [pltpu.SemaphoreType.DMA] * 6
    )

    @functools.partial(
        pl.kernel,
        out_type=(jax.ShapeDtypeStruct((E_pad, 4), F32),
                  jax.ShapeDtypeStruct((E_pad, 4), F32)),
        mesh=_mesh(),
        compiler_params=_SC_PARAMS,
        scratch_types=scratch,
    )
    def k(src_h, dst_h, pos_h, ops_h, opd_h,
          sx0, sx1, dx0, dx1, ps0, ps1, pd0, pd1,
          si0, si1, sg0, sg1, so0, so1):
        sidx = [sx0, sx1]
        didx = [dx0, dx1]
        ps = [ps0, ps1]
        pd = [pd0, pd1]
        semi = [si0, si1]
        semg = [sg0, sg1]
        semo = [so0, so1]

        cid = lax.axis_index("c")
        sid = lax.axis_index("s")
        wid = sid * NCORES + cid
        base = wid * EW

        def issue_idx(b, c):
            off = base + c * CHUNK
            pltpu.async_copy(src_h.at[pl.ds(off, CHUNK)], sidx[b], semi[b])
            pltpu.async_copy(dst_h.at[pl.ds(off, CHUNK)], didx[b], semi[b])

        def wait_idx(b):
            pltpu.make_async_copy(src_h.at[pl.ds(0, CHUNK)], sidx[b], semi[b]).wait()
            pltpu.make_async_copy(dst_h.at[pl.ds(0, CHUNK)], didx[b], semi[b]).wait()

        def drain_out(b):
            pltpu.make_async_copy(ps[b].at[:, 0:4],
                                  ops_h.at[pl.ds(0, CHUNK), :], semo[b]).wait()
            pltpu.make_async_copy(pd[b].at[:, 0:4],
                                  opd_h.at[pl.ds(0, CHUNK), :], semo[b]).wait()

        for b in range(2):
            issue_idx(b, b)

        @pl.loop(0, NCH, step=2)
        def _(g):
            for b in range(2):
                c = g + b
                wait_idx(b)

                @pl.when(c >= 2)
                def _():
                    drain_out(b)

                pltpu.async_copy(pos_h.at[sidx[b]], ps[b], semg[b])
                pltpu.async_copy(pos_h.at[didx[b]], pd[b], semg[b])
                pltpu.make_async_copy(pos_h.at[sidx[b]], ps[b], semg[b]).wait()
                pltpu.make_async_copy(pos_h.at[didx[b]], pd[b], semg[b]).wait()
                pltpu.async_copy(ps[b].at[:, 0:4],
                                 ops_h.at[pl.ds(base + c * CHUNK, CHUNK), :],
                                 semo[b])
                pltpu.async_copy(pd[b].at[:, 0:4],
                                 opd_h.at[pl.ds(base + c * CHUNK, CHUNK), :],
                                 semo[b])

                @pl.when(c + 2 < NCH)
                def _():
                    issue_idx(b, c + 2)

        for b in range(2):
            drain_out(b)

    return k


def _sc_gather(E_pad, W):
    EW = E_pad // NW
    NCH = EW // CHUNK

    scratch = (
        [pltpu.VMEM((CHUNK,), I32) for _ in range(2)]
        + [pltpu.VMEM((CHUNK, W), F32) for _ in range(2)]
        + [pltpu.SemaphoreType.DMA] * 6
    )

    @functools.partial(
        pl.kernel,
        out_type=jax.ShapeDtypeStruct((E_pad, W), F32),
        mesh=_mesh(),
        compiler_params=_SC_PARAMS,
        scratch_types=scratch,
    )
    def k(src_h, tab_h, out_h,
          sx0, sx1, gb0, gb1, si0, si1, sg0, sg1, so0, so1):
        sidx = [sx0, sx1]
        gb = [gb0, gb1]
        semi = [si0, si1]
        semg = [sg0, sg1]
        semo = [so0, so1]

        cid = lax.axis_index("c")
        sid = lax.axis_index("s")
        wid = sid * NCORES + cid
        base = wid * EW

        def issue_idx(b, c):
            pltpu.async_copy(src_h.at[pl.ds(base + c * CHUNK, CHUNK)],
                             sidx[b], semi[b])

        def wait_idx(b):
            pltpu.make_async_copy(src_h.at[pl.ds(0, CHUNK)], sidx[b], semi[b]).wait()

        for b in range(2):
            issue_idx(b, b)

        @pl.loop(0, NCH, step=2)
        def _(g):
            for b in range(2):
                c = g + b
                wait_idx(b)

                @pl.when(c >= 2)
                def _():
                    pltpu.make_async_copy(
                        gb[b], out_h.at[pl.ds(0, CHUNK), :], semo[b]).wait()

                pltpu.async_copy(tab_h.at[sidx[b]], gb[b], semg[b])
                pltpu.make_async_copy(tab_h.at[sidx[b]], gb[b], semg[b]).wait()
                pltpu.async_copy(gb[b],
                                 out_h.at[pl.ds(base + c * CHUNK, CHUNK), :],
                                 semo[b])

                @pl.when(c + 2 < NCH)
                def _():
                    issue_idx(b, c + 2)

        for b in range(2):
            pltpu.make_async_copy(gb[b], out_h.at[pl.ds(0, CHUNK), :], semo[b]).wait()

    return k


def _sc_scatter(E_pad, NP, mode):
    SR = NP // NSUB
    SRC = SR // CHUNK

    if mode == "pair":
        ET = E_pad // NSUB
        n_planes = 4
    else:
        ET = E_pad // NW
        n_planes = 2
    NCH = ET // CHUNK

    scratch = (
        [pltpu.VMEM((CHUNK,), I32) for _ in range(2)]
        + [pltpu.VMEM((CHUNK, 16), F32) for _ in range(2)]
        + [pltpu.VMEM((CHUNK, 16), F32)]
        + [pltpu.VMEM_SHARED((NP, 16), F32)]
        + [pltpu.SemaphoreType.DMA] * 2
    )

    @functools.partial(
        pl.kernel,
        out_type=jax.ShapeDtypeStruct((n_planes * NP, 16), F32),
        mesh=_mesh(),
        compiler_params=_SC_PARAMS,
        scratch_types=scratch,
    )
    def k(dst_h, m_h, agg_h, dx0, dx1, mb0, mb1, zb, acc, sl0, sl1):
        didx = [dx0, dx1]
        mb = [mb0, mb1]
        seml = [sl0, sl1]

        cid = lax.axis_index("c")
        sid = lax.axis_index("s")

        for i in range(CHUNK):
            zb[i, :] = jnp.zeros((LANES,), F32)

        n_comp_loc = 2 if mode == "pair" else 1
        for t in range(n_comp_loc):
            if mode == "pair":
                comp = cid * 2 + t
                ebase = sid * ET
                mrow = comp * E_pad + ebase
                plane = comp
            else:
                comp = cid
                ebase = (sid * NCORES + cid) * ET
                mrow = ebase
                plane = cid

            for kk in range(SRC):
                pltpu.sync_copy(zb, acc.at[pl.ds(sid * SR + kk * CHUNK, CHUNK), :])
            plsc.subcore_barrier()

            def issue_loads(b, c):
                pltpu.async_copy(dst_h.at[pl.ds(ebase + c * CHUNK, CHUNK)],
                                 didx[b], seml[b])
                pltpu.async_copy(m_h.at[pl.ds(mrow + c * CHUNK, CHUNK), :],
                                 mb[b], seml[b])

            def wait_loads(b):
                pltpu.make_async_copy(dst_h.at[pl.ds(0, CHUNK)], didx[b],
                                      seml[b]).wait()
                pltpu.make_async_copy(m_h.at[pl.ds(0, CHUNK), :], mb[b],
                                      seml[b]).wait()

            for b in range(2):
                issue_loads(b, b)

            @pl.loop(0, NCH, step=2)
            def _(g):
                for b in range(2):
                    c = g + b
                    wait_loads(b)
                    pltpu.sync_copy(mb[b], acc.at[didx[b]], add=True)

                    @pl.when(c + 2 < NCH)
                    def _():
                        issue_loads(b, c + 2)

            plsc.subcore_barrier()
            pltpu.sync_copy(
                acc.at[pl.ds(sid * SR, SR), :],
                agg_h.at[pl.ds(plane * NP + sid * SR, SR), :])
            plsc.subcore_barrier()

    return k


def _tc_embed(N, NT):
    grid = (N // BN,)

    def body(a_ref, we_ref, wl_ref, out_ref):
        an = a_ref[0, 0]
        oh = (an[:, None] == lax.broadcasted_iota(I32, (BN, NT), 1)).astype(F32)
        chem = jnp.dot(oh, we_ref[...], preferred_element_type=F32)
        out_ref[...] = jnp.dot(chem, wl_ref[...], preferred_element_type=F32)

    return pl.pallas_call(
        body,
        grid=grid,
        in_specs=[
            pl.BlockSpec((1, 1, BN), lambda i: (i, 0, 0)),
            pl.BlockSpec((NT, 8), lambda i: (0, 0)),
            pl.BlockSpec((8, N_CH), lambda i: (0, 0)),
        ],
        out_specs=pl.BlockSpec((BN, N_CH), lambda i: (i, 0)),
        out_shape=jax.ShapeDtypeStruct((N, N_CH), F32),
    )


def _tc_messages(E_pad, first, last):
    grid = (E_pad // BE,)
    W = 16 if first else 64
    n_out = 1 if last else 4

    def body(ps_ref, pd_ref, g_ref, r1_ref, r2_ref, r3_ref, out_ref):
        d = ps_ref[...][:, 0:3] - pd_ref[...][:, 0:3]
        r2s = jnp.sum(d * d, axis=1, keepdims=True) + 1e-12
        r = jnp.sqrt(r2s)
        inv_r = 1.0 / r
        nvec = (lax.broadcasted_iota(I32, (1, NUM_BASIS), 1) + 1
                ).astype(F32) * (math.pi / R_MAX)
        b = jnp.sin(r * nvec) * (PREF * inv_r)
        x = r * (1.0 / R_MAX)
        x2 = x * x
        x4 = x2 * x2
        x6 = x4 * x2
        x7 = x6 * x
        x8 = x7 * x
        fc = 1.0 - 28.0 * x6 + 48.0 * x7 - 21.0 * x8
        fc = jnp.where(x < 1.0, fc, 0.0)
        basis = b * fc

        h = _silu(jnp.dot(basis, r1_ref[...], preferred_element_type=F32))
        h = _silu(jnp.dot(h, r2_ref[...], preferred_element_type=F32))
        w = jnp.dot(h, r3_ref[...], preferred_element_type=F32)
        w0 = w[:, 0:16]
        w1 = w[:, 16:32]
        w2 = w[:, 32:48]
        w3 = w[:, 48:64]

        y = d * (SQRT3 * inv_r)
        yx = y[:, 0:1]
        yy = y[:, 1:2]
        yz = y[:, 2:3]

        g = g_ref[...]
        s_j = g[:, 0:16]
        if first:
            out_ref[0] = (w0 * s_j) * INV_AVG
            t = w2 * s_j
            out_ref[1] = (t * yx) * INV_AVG
            out_ref[2] = (t * yy) * INV_AVG
            out_ref[3] = (t * yz) * INV_AVG
        else:
            vx = g[:, 16:32]
            vy = g[:, 32:48]
            vz = g[:, 48:64]
            dot = vx * yx + vy * yy + vz * yz
            m_s = (w0 * s_j + w1 * dot) * INV_AVG
            out_ref[0] = m_s
            if not last:
                t = w2 * s_j
                out_ref[1] = (t * yx + w3 * vx) * INV_AVG
                out_ref[2] = (t * yy + w3 * vy) * INV_AVG
                out_ref[3] = (t * yz + w3 * vz) * INV_AVG

    return pl.pallas_call(
        body,
        grid=grid,
        in_specs=[
            pl.BlockSpec((BE, 4), lambda i: (i, 0)),
            pl.BlockSpec((BE, 4), lambda i: (i, 0)),
            pl.BlockSpec((BE, W), lambda i: (i, 0)),
            pl.BlockSpec((NUM_BASIS, 64), lambda i: (0, 0)),
            pl.BlockSpec((64, 64), lambda i: (0, 0)),
            pl.BlockSpec((64, 64), lambda i: (0, 0)),
        ],
        out_specs=pl.BlockSpec((n_out, BE, 16), lambda i: (0, i, 0)),
        out_shape=jax.ShapeDtypeStruct((n_out, E_pad, 16), F32),
    )


def _tc_node_first(N, NP):
    grid = (N // BN,)

    def body(agg_ref, s_ref, wss_ref, wscs_ref, wsv_ref, out_ref):
        a = agg_ref[...]
        s = s_ref[...]
        pre_s = (jnp.dot(a[0], wss_ref[...], preferred_element_type=F32)
                 + jnp.dot(s, wscs_ref[...], preferred_element_type=F32))
        sg = jax.nn.sigmoid(pre_s)
        s1 = pre_s * sg
        wsv = wsv_ref[...]
        vx = jnp.dot(a[1], wsv, preferred_element_type=F32) * sg
        vy = jnp.dot(a[2], wsv, preferred_element_type=F32) * sg
        vz = jnp.dot(a[3], wsv, preferred_element_type=F32) * sg
        out_ref[...] = jnp.concatenate([s1, vx, vy, vz], axis=1)

    return pl.pallas_call(
        body,
        grid=grid,
        in_specs=[
            pl.BlockSpec((4, BN, 16), lambda i: (0, i, 0)),
            pl.BlockSpec((BN, 16), lambda i: (i, 0)),
            pl.BlockSpec((16, 16), lambda i: (0, 0)),
            pl.BlockSpec((16, 16), lambda i: (0, 0)),
            pl.BlockSpec((16, 16), lambda i: (0, 0)),
        ],
        out_specs=pl.BlockSpec((BN, 64), lambda i: (i, 0)),
        out_shape=jax.ShapeDtypeStruct((N, 64), F32),
    )


def _tc_node_mid(N, NP):
    grid = (N // BN,)

    def body(agg_ref, st_ref, wss_ref, wscs_ref, wsv_ref, wscv_ref, out_ref):
        a = agg_ref[...]
        st = st_ref[...]
        s = st[:, 0:16]
        pre_s = (jnp.dot(a[0], wss_ref[...], preferred_element_type=F32)
                 + jnp.dot(s, wscs_ref[...], preferred_element_type=F32))
        sg = jax.nn.sigmoid(pre_s)
        s1 = pre_s * sg
        wsv = wsv_ref[...]
        wscv = wscv_ref[...]
        vx = (jnp.dot(a[1], wsv, preferred_element_type=F32)
              + jnp.dot(st[:, 16:32], wscv, preferred_element_type=F32)) * sg
        vy = (jnp.dot(a[2], wsv, preferred_element_type=F32)
              + jnp.dot(st[:, 32:48], wscv, preferred_element_type=F32)) * sg
        vz = (jnp.dot(a[3], wsv, preferred_element_type=F32)
              + jnp.dot(st[:, 48:64], wscv, preferred_element_type=F32)) * sg
        out_ref[...] = jnp.concatenate([s1, vx, vy, vz], axis=1)

    return pl.pallas_call(
        body,
        grid=grid,
        in_specs=[
            pl.BlockSpec((4, BN, 16), lambda i: (0, i, 0)),
            pl.BlockSpec((BN, 64), lambda i: (i, 0)),
            pl.BlockSpec((16, 16), lambda i: (0, 0)),
            pl.BlockSpec((16, 16), lambda i: (0, 0)),
            pl.BlockSpec((16, 16), lambda i: (0, 0)),
            pl.BlockSpec((16, 16), lambda i: (0, 0)),
        ],
        out_specs=pl.BlockSpec((BN, 64), lambda i: (i, 0)),
        out_shape=jax.ShapeDtypeStruct((N, 64), F32),
    )


def _tc_node_last(N, NP, NG):
    grid = (N // BN,)

    def body(agg_ref, st_ref, wss_ref, wscs_ref, w1_ref, w2_ref, b_ref,
             out_ref):
        a = agg_ref[...]
        agg_s = a[0] + a[1]
        s = st_ref[...][:, 0:16]
        pre_s = (jnp.dot(agg_s, wss_ref[...], preferred_element_type=F32)
                 + jnp.dot(s, wscs_ref[...], preferred_element_type=F32))
        s2 = pre_s * jax.nn.sigmoid(pre_s)
        h = jnp.dot(s2, w1_ref[...], preferred_element_type=F32)
        pa = jnp.dot(h, w2_ref[...], preferred_element_type=F32)
        bid = b_ref[0, 0]
        oh = (bid[:, None] == lax.broadcasted_iota(I32, (BN, NG), 1)).astype(F32)
        e = jnp.sum(oh * pa, axis=0)

        @pl.when(pl.program_id(0) == 0)
        def _():
            out_ref[...] = jnp.zeros_like(out_ref)

        out_ref[...] += e[None, :]

    return pl.pallas_call(
        body,
        grid=grid,
        in_specs=[
            pl.BlockSpec((2, BN, 16), lambda i: (0, i, 0)),
            pl.BlockSpec((BN, 64), lambda i: (i, 0)),
            pl.BlockSpec((16, 16), lambda i: (0, 0)),
            pl.BlockSpec((16, 16), lambda i: (0, 0)),
            pl.BlockSpec((16, 8), lambda i: (0, 0)),
            pl.BlockSpec((8, 1), lambda i: (0, 0)),
            pl.BlockSpec((1, 1, BN), lambda i: (i, 0, 0)),
        ],
        out_specs=pl.BlockSpec((1, NG), lambda i: (0, 0)),
        out_shape=jax.ShapeDtypeStruct((1, NG), F32),
    )


def kernel(pos, atomic_numbers, batch, edge_index, W_embed, W_lift,
           R1, R2, R3, W_self_s, W_self_v, W_sc_s, W_sc_v, W_out1, W_out2):
    N = pos.shape[0]
    E = edge_index.shape[1]
    NT = W_embed.shape[0]
    NG = 64

    E_pad = _ru(E, NW * CHUNK * 2)
    NP = _ru(N + 1, NSUB * CHUNK)

    src = jnp.concatenate(
        [edge_index[0].astype(I32), jnp.zeros((E_pad - E,), I32)])
    dst = jnp.concatenate(
        [edge_index[1].astype(I32), jnp.full((E_pad - E,), N, I32)])
    pos4 = jnp.zeros((N + 8, 16), F32).at[:N, :3].set(pos.astype(F32))
    atom3 = atomic_numbers.astype(I32).reshape(N // BN, 1, BN)
    batch3 = batch.astype(I32).reshape(N // BN, 1, BN)

    s0 = _tc_embed(N, NT)(atom3, W_embed, W_lift)
    ps4, pd4 = _sc_edge_vec(E_pad, N + 8)(src, dst, pos4)

    g0 = _sc_gather(E_pad, 16)(src, s0)
    m0 = _tc_messages(E_pad, first=True, last=False)(ps4, pd4, g0, R1[0], R2[0], R3[0])
    agg0 = _sc_scatter(E_pad, NP, "pair")(dst, m0.reshape(4 * E_pad, 16))
    st1 = _tc_node_first(N, NP)(
        agg0.reshape(4, NP, 16), s0, W_self_s[0], W_sc_s[0], W_self_v[0])

    g1 = _sc_gather(E_pad, 64)(src, st1)
    m1 = _tc_messages(E_pad, first=False, last=False)(ps4, pd4, g1, R1[1], R2[1], R3[1])
    agg1 = _sc_scatter(E_pad, NP, "pair")(dst, m1.reshape(4 * E_pad, 16))
    st2 = _tc_node_mid(N, NP)(
        agg1.reshape(4, NP, 16), st1, W_self_s[1], W_sc_s[1], W_self_v[1],
        W_sc_v[1])

    g2 = _sc_gather(E_pad, 64)(src, st2)
    m2 = _tc_messages(E_pad, first=False, last=True)(ps4, pd4, g2, R1[2], R2[2], R3[2])
    agg2 = _sc_scatter(E_pad, NP, "half")(dst, m2.reshape(E_pad, 16))
    energy = _tc_node_last(N, NP, NG)(
        agg2.reshape(2, NP, 16), st2, W_self_s[2], W_sc_s[2], W_out1, W_out2,
        batch3)

    return energy.reshape(NG)


# speedup vs baseline: 15.7143x; 1.2225x over previous
import functools
import math

import jax
import jax.numpy as jnp
from jax import lax
from jax.experimental import pallas as pl
from jax.experimental.pallas import tpu as pltpu
from jax.experimental.pallas import tpu_sc as plsc

F32 = jnp.float32
I32 = jnp.int32

NCORES = 2
NSUB = 16
NW = NCORES * NSUB
LANES = 16
CHUNK = 128

R_MAX = 4.0
NUM_BASIS = 8
P_CUT = 6.0
N_CH = 16
INV_AVG = 1.0 / 16.0
SQRT3 = math.sqrt(3.0)
PREF = math.sqrt(2.0 / R_MAX)

BE = 4096
BN = 2000


def _ru(x, m):
    return (x + m - 1) // m * m


def _mesh():
    return plsc.VectorSubcoreMesh(
        core_axis_name="c", subcore_axis_name="s",
        num_cores=NCORES, num_subcores=NSUB)


_SC_PARAMS = pltpu.CompilerParams(use_tc_tiling_on_sc=False)


def _silu(x):
    return x * jax.nn.sigmoid(x)


def _sc_edge_vec(E_pad, NPOS):
    EW = E_pad // NW
    NCH = EW // CHUNK

    scratch = (
        [pltpu.VMEM((CHUNK,), I32) for _ in range(4)]
        + [pltpu.VMEM((CHUNK, 16), F32) for _ in range(6)]
        + [pltpu.SemaphoreType.DMA] * 6
    )

    @functools.partial(
        pl.kernel,
        out_type=jax.ShapeDtypeStruct((E_pad, 16), F32),
        mesh=_mesh(),
        compiler_params=_SC_PARAMS,
        scratch_types=scratch,
    )
    def k(src_h, dst_h, pos_h, ev_h,
          sx0, sx1, dx0, dx1, ps0, ps1, pd0, pd1, db0, db1,
          si0, si1, sg0, sg1, so0, so1):
        sidx = [sx0, sx1]
        didx = [dx0, dx1]
        ps = [ps0, ps1]
        pd = [pd0, pd1]
        db = [db0, db1]
        semi = [si0, si1]
        semg = [sg0, sg1]
        semo = [so0, so1]

        cid = lax.axis_index("c")
        sid = lax.axis_index("s")
        wid = sid * NCORES + cid
        base = wid * EW

        def issue_idx(b, c):
            off = base + c * CHUNK
            pltpu.async_copy(src_h.at[pl.ds(off, CHUNK)], sidx[b], semi[b])
            pltpu.async_copy(dst_h.at[pl.ds(off, CHUNK)], didx[b], semi[b])

        def wait_idx(b):
            pltpu.make_async_copy(src_h.at[pl.ds(0, CHUNK)], sidx[b], semi[b]).wait()
            pltpu.make_async_copy(dst_h.at[pl.ds(0, CHUNK)], didx[b], semi[b]).wait()

        def drain_out(b):
            pltpu.make_async_copy(db[b], ev_h.at[pl.ds(0, CHUNK), :], semo[b]).wait()

        for b in range(2):
            issue_idx(b, b)

        @pl.loop(0, NCH, step=2)
        def _(g):
            for b in range(2):
                c = g + b
                wait_idx(b)

                pltpu.async_copy(pos_h.at[sidx[b]], ps[b], semg[b])
                pltpu.async_copy(pos_h.at[didx[b]], pd[b], semg[b])
                pltpu.make_async_copy(pos_h.at[sidx[b]], ps[b], semg[b]).wait()
                pltpu.make_async_copy(pos_h.at[didx[b]], pd[b], semg[b]).wait()

                @pl.when(c >= 2)
                def _():
                    drain_out(b)

                for i in range(CHUNK):
                    db[b][i, :] = ps[b][i, :] - pd[b][i, :]
                pltpu.async_copy(db[b],
                                 ev_h.at[pl.ds(base + c * CHUNK, CHUNK), :],
                                 semo[b])

                @pl.when(c + 2 < NCH)
                def _():
                    issue_idx(b, c + 2)

        for b in range(2):
            drain_out(b)

    return k


def _sc_gather(E_pad, W):
    EW = E_pad // NW
    NCH = EW // CHUNK

    scratch = (
        [pltpu.VMEM((CHUNK,), I32) for _ in range(2)]
        + [pltpu.VMEM((CHUNK, W), F32) for _ in range(2)]
        + [pltpu.SemaphoreType.DMA] * 6
    )

    @functools.partial(
        pl.kernel,
        out_type=jax.ShapeDtypeStruct((E_pad, W), F32),
        mesh=_mesh(),
        compiler_params=_SC_PARAMS,
        scratch_types=scratch,
    )
    def k(src_h, tab_h, out_h,
          sx0, sx1, gb0, gb1, si0, si1, sg0, sg1, so0, so1):
        sidx = [sx0, sx1]
        gb = [gb0, gb1]
        semi = [si0, si1]
        semg = [sg0, sg1]
        semo = [so0, so1]

        cid = lax.axis_index("c")
        sid = lax.axis_index("s")
        wid = sid * NCORES + cid
        base = wid * EW

        def issue_idx(b, c):
            pltpu.async_copy(src_h.at[pl.ds(base + c * CHUNK, CHUNK)],
                             sidx[b], semi[b])

        def wait_idx(b):
            pltpu.make_async_copy(src_h.at[pl.ds(0, CHUNK)], sidx[b], semi[b]).wait()

        for b in range(2):
            issue_idx(b, b)

        @pl.loop(0, NCH, step=2)
        def _(g):
            for b in range(2):
                c = g + b
                wait_idx(b)

                @pl.when(c >= 2)
                def _():
                    pltpu.make_async_copy(
                        gb[b], out_h.at[pl.ds(0, CHUNK), :], semo[b]).wait()

                pltpu.async_copy(tab_h.at[sidx[b]], gb[b], semg[b])
                pltpu.make_async_copy(tab_h.at[sidx[b]], gb[b], semg[b]).wait()
                pltpu.async_copy(gb[b],
                                 out_h.at[pl.ds(base + c * CHUNK, CHUNK), :],
                                 semo[b])

                @pl.when(c + 2 < NCH)
                def _():
                    issue_idx(b, c + 2)

        for b in range(2):
            pltpu.make_async_copy(gb[b], out_h.at[pl.ds(0, CHUNK), :], semo[b]).wait()

    return k


def _sc_scatter(E_pad, NP, mode):
    SR = NP // NSUB
    SRC = SR // CHUNK

    if mode == "pair":
        ET = E_pad // NSUB
        n_planes = 4
        n_comp = 4
    else:
        ET = E_pad // NW
        n_planes = 2
        n_comp = 1
    NCH = ET // CHUNK

    scratch = (
        [pltpu.VMEM((CHUNK,), I32) for _ in range(2)]
        + [pltpu.VMEM((CHUNK, 16), F32) for _ in range(2)]
        + [pltpu.VMEM((CHUNK, 16), F32)]
        + [pltpu.VMEM_SHARED((NP, 16), F32)]
        + [pltpu.SemaphoreType.DMA] * 2
    )

    @functools.partial(
        pl.kernel,
        out_type=jax.ShapeDtypeStruct((n_planes * NP, 16), F32),
        mesh=_mesh(),
        compiler_params=_SC_PARAMS,
        scratch_types=scratch,
    )
    def k(dst_h, m_h, agg_h, dx0, dx1, mb0, mb1, zb, acc, sl0, sl1):
        didx = [dx0, dx1]
        mb = [mb0, mb1]
        seml = [sl0, sl1]

        cid = lax.axis_index("c")
        sid = lax.axis_index("s")

        for i in range(CHUNK):
            zb[i, :] = jnp.zeros((LANES,), F32)

        n_comp_loc = 2 if mode == "pair" else 1
        for t in range(n_comp_loc):
            if mode == "pair":
                comp = cid * 2 + t
                ebase = sid * ET
                plane = comp
            else:
                comp = 0
                ebase = (sid * NCORES + cid) * ET
                plane = cid

            for kk in range(SRC):
                pltpu.sync_copy(zb, acc.at[pl.ds(sid * SR + kk * CHUNK, CHUNK), :])
            plsc.subcore_barrier()

            def issue_loads(b, c):
                pltpu.async_copy(dst_h.at[pl.ds(ebase + c * CHUNK, CHUNK)],
                                 didx[b], seml[b])
                pltpu.async_copy(m_h.at[comp, pl.ds(ebase + c * CHUNK, CHUNK), :],
                                 mb[b], seml[b])

            def wait_loads(b):
                pltpu.make_async_copy(dst_h.at[pl.ds(0, CHUNK)], didx[b],
                                      seml[b]).wait()
                pltpu.make_async_copy(m_h.at[0, pl.ds(0, CHUNK), :], mb[b],
                                      seml[b]).wait()

            for b in range(2):
                issue_loads(b, b)

            @pl.loop(0, NCH, step=2)
            def _(g):
                for b in range(2):
                    c = g + b
                    wait_loads(b)
                    pltpu.sync_copy(mb[b], acc.at[didx[b]], add=True)

                    @pl.when(c + 2 < NCH)
                    def _():
                        issue_loads(b, c + 2)

            plsc.subcore_barrier()
            pltpu.sync_copy(
                acc.at[pl.ds(sid * SR, SR), :],
                agg_h.at[pl.ds(plane * NP + sid * SR, SR), :])
            plsc.subcore_barrier()

    return k


def _tc_embed(N, NT):
    grid = (N // BN,)

    def body(a_ref, we_ref, wl_ref, out_ref):
        an = a_ref[0, 0]
        oh = (an[:, None] == lax.broadcasted_iota(I32, (BN, NT), 1)).astype(F32)
        chem = jnp.dot(oh, we_ref[...], preferred_element_type=F32)
        out_ref[...] = jnp.dot(chem, wl_ref[...], preferred_element_type=F32)

    return pl.pallas_call(
        body,
        grid=grid,
        in_specs=[
            pl.BlockSpec((1, 1, BN), lambda i: (i, 0, 0)),
            pl.BlockSpec((NT, 8), lambda i: (0, 0)),
            pl.BlockSpec((8, N_CH), lambda i: (0, 0)),
        ],
        out_specs=pl.BlockSpec((BN, N_CH), lambda i: (i, 0)),
        out_shape=jax.ShapeDtypeStruct((N, N_CH), F32),
    )


def _tc_messages(E_pad, first, last):
    grid = (E_pad // BE,)
    W = 16 if first else 64
    n_out = 1 if last else 4

    def body(ev_ref, g_ref, r1_ref, r2_ref, r3_ref, out_ref):
        d = ev_ref[...][:, 0:3]
        r2s = jnp.sum(d * d, axis=1, keepdims=True) + 1e-12
        r = jnp.sqrt(r2s)
        inv_r = 1.0 / r
        nvec = (lax.broadcasted_iota(I32, (1, NUM_BASIS), 1) + 1
                ).astype(F32) * (math.pi / R_MAX)
        b = jnp.sin(r * nvec) * (PREF * inv_r)
        x = r * (1.0 / R_MAX)
        x2 = x * x
        x4 = x2 * x2
        x6 = x4 * x2
        x7 = x6 * x
        x8 = x7 * x
        fc = 1.0 - 28.0 * x6 + 48.0 * x7 - 21.0 * x8
        fc = jnp.where(x < 1.0, fc, 0.0)
        basis = b * fc

        h = _silu(jnp.dot(basis, r1_ref[...], preferred_element_type=F32))
        h = _silu(jnp.dot(h, r2_ref[...], preferred_element_type=F32))
        w = jnp.dot(h, r3_ref[...], preferred_element_type=F32)
        w0 = w[:, 0:16]
        w1 = w[:, 16:32]
        w2 = w[:, 32:48]
        w3 = w[:, 48:64]

        y = d * (SQRT3 * inv_r)
        yx = y[:, 0:1]
        yy = y[:, 1:2]
        yz = y[:, 2:3]

        g = g_ref[...]
        s_j = g[:, 0:16]
        if first:
            out_ref[0] = (w0 * s_j) * INV_AVG
            t = w2 * s_j
            out_ref[1] = (t * yx) * INV_AVG
            out_ref[2] = (t * yy) * INV_AVG
            out_ref[3] = (t * yz) * INV_AVG
        else:
            vx = g[:, 16:32]
            vy = g[:, 32:48]
            vz = g[:, 48:64]
            dot = vx * yx + vy * yy + vz * yz
            m_s = (w0 * s_j + w1 * dot) * INV_AVG
            out_ref[0] = m_s
            if not last:
                t = w2 * s_j
                out_ref[1] = (t * yx + w3 * vx) * INV_AVG
                out_ref[2] = (t * yy + w3 * vy) * INV_AVG
                out_ref[3] = (t * yz + w3 * vz) * INV_AVG

    return pl.pallas_call(
        body,
        grid=grid,
        in_specs=[
            pl.BlockSpec((BE, 16), lambda i: (i, 0)),
            pl.BlockSpec((BE, W), lambda i: (i, 0)),
            pl.BlockSpec((NUM_BASIS, 64), lambda i: (0, 0)),
            pl.BlockSpec((64, 64), lambda i: (0, 0)),
            pl.BlockSpec((64, 64), lambda i: (0, 0)),
        ],
        out_specs=pl.BlockSpec((n_out, BE, 16), lambda i: (0, i, 0)),
        out_shape=jax.ShapeDtypeStruct((n_out, E_pad, 16), F32),
    )


def _tc_node_first(N, NP):
    grid = (N // BN,)

    def body(agg_ref, s_ref, wss_ref, wscs_ref, wsv_ref, out_ref):
        a = agg_ref[...]
        s = s_ref[...]
        pre_s = (jnp.dot(a[0], wss_ref[...], preferred_element_type=F32)
                 + jnp.dot(s, wscs_ref[...], preferred_element_type=F32))
        sg = jax.nn.sigmoid(pre_s)
        s1 = pre_s * sg
        wsv = wsv_ref[...]
        vx = jnp.dot(a[1], wsv, preferred_element_type=F32) * sg
        vy = jnp.dot(a[2], wsv, preferred_element_type=F32) * sg
        vz = jnp.dot(a[3], wsv, preferred_element_type=F32) * sg
        out_ref[...] = jnp.concatenate([s1, vx, vy, vz], axis=1)

    return pl.pallas_call(
        body,
        grid=grid,
        in_specs=[
            pl.BlockSpec((4, BN, 16), lambda i: (0, i, 0)),
            pl.BlockSpec((BN, 16), lambda i: (i, 0)),
            pl.BlockSpec((16, 16), lambda i: (0, 0)),
            pl.BlockSpec((16, 16), lambda i: (0, 0)),
            pl.BlockSpec((16, 16), lambda i: (0, 0)),
        ],
        out_specs=pl.BlockSpec((BN, 64), lambda i: (i, 0)),
        out_shape=jax.ShapeDtypeStruct((N, 64), F32),
    )


def _tc_node_mid(N, NP):
    grid = (N // BN,)

    def body(agg_ref, st_ref, wss_ref, wscs_ref, wsv_ref, wscv_ref, out_ref):
        a = agg_ref[...]
        st = st_ref[...]
        s = st[:, 0:16]
        pre_s = (jnp.dot(a[0], wss_ref[...], preferred_element_type=F32)
                 + jnp.dot(s, wscs_ref[...], preferred_element_type=F32))
        sg = jax.nn.sigmoid(pre_s)
        s1 = pre_s * sg
        wsv = wsv_ref[...]
        wscv = wscv_ref[...]
        vx = (jnp.dot(a[1], wsv, preferred_element_type=F32)
              + jnp.dot(st[:, 16:32], wscv, preferred_element_type=F32)) * sg
        vy = (jnp.dot(a[2], wsv, preferred_element_type=F32)
              + jnp.dot(st[:, 32:48], wscv, preferred_element_type=F32)) * sg
        vz = (jnp.dot(a[3], wsv, preferred_element_type=F32)
              + jnp.dot(st[:, 48:64], wscv, preferred_element_type=F32)) * sg
        out_ref[...] = jnp.concatenate([s1, vx, vy, vz], axis=1)

    return pl.pallas_call(
        body,
        grid=grid,
        in_specs=[
            pl.BlockSpec((4, BN, 16), lambda i: (0, i, 0)),
            pl.BlockSpec((BN, 64), lambda i: (i, 0)),
            pl.BlockSpec((16, 16), lambda i: (0, 0)),
            pl.BlockSpec((16, 16), lambda i: (0, 0)),
            pl.BlockSpec((16, 16), lambda i: (0, 0)),
            pl.BlockSpec((16, 16), lambda i: (0, 0)),
        ],
        out_specs=pl.BlockSpec((BN, 64), lambda i: (i, 0)),
        out_shape=jax.ShapeDtypeStruct((N, 64), F32),
    )


def _tc_node_last(N, NP, NG):
    grid = (N // BN,)

    def body(agg_ref, st_ref, wss_ref, wscs_ref, w1_ref, w2_ref, b_ref,
             out_ref):
        a = agg_ref[...]
        agg_s = a[0] + a[1]
        s = st_ref[...][:, 0:16]
        pre_s = (jnp.dot(agg_s, wss_ref[...], preferred_element_type=F32)
                 + jnp.dot(s, wscs_ref[...], preferred_element_type=F32))
        s2 = pre_s * jax.nn.sigmoid(pre_s)
        h = jnp.dot(s2, w1_ref[...], preferred_element_type=F32)
        pa = jnp.dot(h, w2_ref[...], preferred_element_type=F32)
        bid = b_ref[0, 0]
        oh = (bid[:, None] == lax.broadcasted_iota(I32, (BN, NG), 1)).astype(F32)
        e = jnp.sum(oh * pa, axis=0)

        @pl.when(pl.program_id(0) == 0)
        def _():
            out_ref[...] = jnp.zeros_like(out_ref)

        out_ref[...] += e[None, :]

    return pl.pallas_call(
        body,
        grid=grid,
        in_specs=[
            pl.BlockSpec((2, BN, 16), lambda i: (0, i, 0)),
            pl.BlockSpec((BN, 64), lambda i: (i, 0)),
            pl.BlockSpec((16, 16), lambda i: (0, 0)),
            pl.BlockSpec((16, 16), lambda i: (0, 0)),
            pl.BlockSpec((16, 8), lambda i: (0, 0)),
            pl.BlockSpec((8, 1), lambda i: (0, 0)),
            pl.BlockSpec((1, 1, BN), lambda i: (i, 0, 0)),
        ],
        out_specs=pl.BlockSpec((1, NG), lambda i: (0, 0)),
        out_shape=jax.ShapeDtypeStruct((1, NG), F32),
    )


def kernel(pos, atomic_numbers, batch, edge_index, W_embed, W_lift,
           R1, R2, R3, W_self_s, W_self_v, W_sc_s, W_sc_v, W_out1, W_out2):
    N = pos.shape[0]
    E = edge_index.shape[1]
    NT = W_embed.shape[0]
    NG = 64

    E_pad = _ru(E, NW * CHUNK * 2)
    NP = _ru(N + 1, NSUB * CHUNK)

    src = jnp.concatenate(
        [edge_index[0].astype(I32), jnp.zeros((E_pad - E,), I32)])
    dst = jnp.concatenate(
        [edge_index[1].astype(I32), jnp.full((E_pad - E,), N, I32)])
    pos4 = jnp.zeros((N + 8, 16), F32).at[:N, :3].set(pos.astype(F32))
    atom3 = atomic_numbers.astype(I32).reshape(N // BN, 1, BN)
    batch3 = batch.astype(I32).reshape(N // BN, 1, BN)

    s0 = _tc_embed(N, NT)(atom3, W_embed, W_lift)
    ev = _sc_edge_vec(E_pad, N + 8)(src, dst, pos4)

    g0 = _sc_gather(E_pad, 16)(src, s0)
    m0 = _tc_messages(E_pad, first=True, last=False)(ev, g0, R1[0], R2[0], R3[0])
    agg0 = _sc_scatter(E_pad, NP, "pair")(dst, m0)
    st1 = _tc_node_first(N, NP)(
        agg0.reshape(4, NP, 16), s0, W_self_s[0], W_sc_s[0], W_self_v[0])

    g1 = _sc_gather(E_pad, 64)(src, st1)
    m1 = _tc_messages(E_pad, first=False, last=False)(ev, g1, R1[1], R2[1], R3[1])
    agg1 = _sc_scatter(E_pad, NP, "pair")(dst, m1)
    st2 = _tc_node_mid(N, NP)(
        agg1.reshape(4, NP, 16), st1, W_self_s[1], W_sc_s[1], W_self_v[1],
        W_sc_v[1])

    g2 = _sc_gather(E_pad, 64)(src, st2)
    m2 = _tc_messages(E_pad, first=False, last=True)(ev, g2, R1[2], R2[2], R3[2])
    agg2 = _sc_scatter(E_pad, NP, "half")(dst, m2)
    energy = _tc_node_last(N, NP, NG)(
        agg2.reshape(2, NP, 16), st2, W_self_s[2], W_sc_s[2], W_out1, W_out2,
        batch3)

    return energy.reshape(NG)


# speedup vs baseline: 20.1345x; 1.2813x over previous
import functools
import math

import jax
import jax.numpy as jnp
from jax import lax
from jax.experimental import pallas as pl
from jax.experimental.pallas import tpu as pltpu
from jax.experimental.pallas import tpu_sc as plsc

F32 = jnp.float32
I32 = jnp.int32

NCORES = 2
NSUB = 16
NW = NCORES * NSUB
LANES = 16
CHUNK = 128

R_MAX = 4.0
NUM_BASIS = 8
P_CUT = 6.0
N_CH = 16
INV_AVG = 1.0 / 16.0
SQRT3 = math.sqrt(3.0)
PREF = math.sqrt(2.0 / R_MAX)

BE = 4096
BN = 2000


def _ru(x, m):
    return (x + m - 1) // m * m


def _mesh():
    return plsc.VectorSubcoreMesh(
        core_axis_name="c", subcore_axis_name="s",
        num_cores=NCORES, num_subcores=NSUB)


_SC_PARAMS = pltpu.CompilerParams(use_tc_tiling_on_sc=False)


def _silu(x):
    return x * jax.nn.sigmoid(x)


def _sc_edge_vec(E_pad, NPOS):
    EW = E_pad // NW
    NCH = EW // CHUNK

    scratch = (
        [pltpu.VMEM((CHUNK,), I32) for _ in range(4)]
        + [pltpu.VMEM((CHUNK, 16), F32) for _ in range(6)]
        + [pltpu.SemaphoreType.DMA] * 6
    )

    @functools.partial(
        pl.kernel,
        out_type=jax.ShapeDtypeStruct((E_pad, 16), F32),
        mesh=_mesh(),
        compiler_params=_SC_PARAMS,
        scratch_types=scratch,
    )
    def k(src_h, dst_h, pos_h, ev_h,
          sx0, sx1, dx0, dx1, ps0, ps1, pd0, pd1, db0, db1,
          si0, si1, sg0, sg1, so0, so1):
        sidx = [sx0, sx1]
        didx = [dx0, dx1]
        ps = [ps0, ps1]
        pd = [pd0, pd1]
        db = [db0, db1]
        semi = [si0, si1]
        semg = [sg0, sg1]
        semo = [so0, so1]

        cid = lax.axis_index("c")
        sid = lax.axis_index("s")
        wid = sid * NCORES + cid
        base = wid * EW

        def issue_idx(b, c):
            off = base + c * CHUNK
            pltpu.async_copy(src_h.at[pl.ds(off, CHUNK)], sidx[b], semi[b])
            pltpu.async_copy(dst_h.at[pl.ds(off, CHUNK)], didx[b], semi[b])

        def wait_idx(b):
            pltpu.make_async_copy(src_h.at[pl.ds(0, CHUNK)], sidx[b], semi[b]).wait()
            pltpu.make_async_copy(dst_h.at[pl.ds(0, CHUNK)], didx[b], semi[b]).wait()

        def drain_out(b):
            pltpu.make_async_copy(db[b], ev_h.at[pl.ds(0, CHUNK), :], semo[b]).wait()

        for b in range(2):
            issue_idx(b, b)

        @pl.loop(0, NCH, step=2)
        def _(g):
            for b in range(2):
                c = g + b
                wait_idx(b)

                pltpu.async_copy(pos_h.at[sidx[b]], ps[b], semg[b])
                pltpu.async_copy(pos_h.at[didx[b]], pd[b], semg[b])
                pltpu.make_async_copy(pos_h.at[sidx[b]], ps[b], semg[b]).wait()
                pltpu.make_async_copy(pos_h.at[didx[b]], pd[b], semg[b]).wait()

                @pl.when(c >= 2)
                def _():
                    drain_out(b)

                for i in range(CHUNK):
                    db[b][i, :] = ps[b][i, :] - pd[b][i, :]
                pltpu.async_copy(db[b],
                                 ev_h.at[pl.ds(base + c * CHUNK, CHUNK), :],
                                 semo[b])

                @pl.when(c + 2 < NCH)
                def _():
                    issue_idx(b, c + 2)

        for b in range(2):
            drain_out(b)

    return k


def _sc_gather(E_pad, W):
    EW = E_pad // NW
    NCH = EW // CHUNK

    scratch = (
        [pltpu.VMEM((CHUNK,), I32) for _ in range(2)]
        + [pltpu.VMEM((CHUNK, W), F32) for _ in range(2)]
        + [pltpu.SemaphoreType.DMA] * 6
    )

    @functools.partial(
        pl.kernel,
        out_type=jax.ShapeDtypeStruct((E_pad, W), F32),
        mesh=_mesh(),
        compiler_params=_SC_PARAMS,
        scratch_types=scratch,
    )
    def k(src_h, tab_h, out_h,
          sx0, sx1, gb0, gb1, si0, si1, sg0, sg1, so0, so1):
        sidx = [sx0, sx1]
        gb = [gb0, gb1]
        semi = [si0, si1]
        semg = [sg0, sg1]
        semo = [so0, so1]

        cid = lax.axis_index("c")
        sid = lax.axis_index("s")
        wid = sid * NCORES + cid
        base = wid * EW

        def issue_idx(b, c):
            pltpu.async_copy(src_h.at[pl.ds(base + c * CHUNK, CHUNK)],
                             sidx[b], semi[b])

        def wait_idx(b):
            pltpu.make_async_copy(src_h.at[pl.ds(0, CHUNK)], sidx[b], semi[b]).wait()

        for b in range(2):
            issue_idx(b, b)

        @pl.loop(0, NCH, step=2)
        def _(g):
            for b in range(2):
                c = g + b
                wait_idx(b)

                @pl.when(c >= 2)
                def _():
                    pltpu.make_async_copy(
                        gb[b], out_h.at[pl.ds(0, CHUNK), :], semo[b]).wait()

                pltpu.async_copy(tab_h.at[sidx[b]], gb[b], semg[b])
                pltpu.make_async_copy(tab_h.at[sidx[b]], gb[b], semg[b]).wait()
                pltpu.async_copy(gb[b],
                                 out_h.at[pl.ds(base + c * CHUNK, CHUNK), :],
                                 semo[b])

                @pl.when(c + 2 < NCH)
                def _():
                    issue_idx(b, c + 2)

        for b in range(2):
            pltpu.make_async_copy(gb[b], out_h.at[pl.ds(0, CHUNK), :], semo[b]).wait()

    return k


def _sc_scatter(E_pad, NP, mode):
    SR = NP // NSUB
    SRC = SR // CHUNK

    if mode == "pair":
        ET = E_pad // NSUB
        n_planes = 4
        n_comp = 4
    else:
        ET = E_pad // NW
        n_planes = 2
        n_comp = 1
    NCH = ET // CHUNK

    scratch = (
        [pltpu.VMEM((CHUNK,), I32) for _ in range(2)]
        + [pltpu.VMEM((CHUNK, 16), F32) for _ in range(2)]
        + [pltpu.VMEM((CHUNK, 16), F32)]
        + [pltpu.VMEM_SHARED((NP, 16), F32)]
        + [pltpu.SemaphoreType.DMA] * 2
    )

    @functools.partial(
        pl.kernel,
        out_type=jax.ShapeDtypeStruct((n_planes * NP, 16), F32),
        mesh=_mesh(),
        compiler_params=_SC_PARAMS,
        scratch_types=scratch,
    )
    def k(dst_h, m_h, agg_h, dx0, dx1, mb0, mb1, zb, acc, sl0, sl1):
        didx = [dx0, dx1]
        mb = [mb0, mb1]
        seml = [sl0, sl1]

        cid = lax.axis_index("c")
        sid = lax.axis_index("s")

        for i in range(CHUNK):
            zb[i, :] = jnp.zeros((LANES,), F32)

        n_comp_loc = 2 if mode == "pair" else 1
        for t in range(n_comp_loc):
            if mode == "pair":
                comp = cid * 2 + t
                ebase = sid * ET
                plane = comp
            else:
                comp = 0
                ebase = (sid * NCORES + cid) * ET
                plane = cid

            for kk in range(SRC):
                pltpu.sync_copy(zb, acc.at[pl.ds(sid * SR + kk * CHUNK, CHUNK), :])
            plsc.subcore_barrier()

            def issue_loads(b, c):
                pltpu.async_copy(dst_h.at[pl.ds(ebase + c * CHUNK, CHUNK)],
                                 didx[b], seml[b])
                pltpu.async_copy(m_h.at[comp, pl.ds(ebase + c * CHUNK, CHUNK), :],
                                 mb[b], seml[b])

            def wait_loads(b):
                pltpu.make_async_copy(dst_h.at[pl.ds(0, CHUNK)], didx[b],
                                      seml[b]).wait()
                pltpu.make_async_copy(m_h.at[0, pl.ds(0, CHUNK), :], mb[b],
                                      seml[b]).wait()

            for b in range(2):
                issue_loads(b, b)

            @pl.loop(0, NCH, step=2)
            def _(g):
                for b in range(2):
                    c = g + b
                    wait_loads(b)
                    pltpu.sync_copy(mb[b], acc.at[didx[b]], add=True)

                    @pl.when(c + 2 < NCH)
                    def _():
                        issue_loads(b, c + 2)

            plsc.subcore_barrier()
            pltpu.sync_copy(
                acc.at[pl.ds(sid * SR, SR), :],
                agg_h.at[pl.ds(plane * NP + sid * SR, SR), :])
            plsc.subcore_barrier()

    return k


def _tc_embed(N, NT):
    grid = (N // BN,)

    def body(a_ref, we_ref, wl_ref, out_ref):
        an = a_ref[0, 0]
        oh = (an[:, None] == lax.broadcasted_iota(I32, (BN, NT), 1)).astype(F32)
        chem = jnp.dot(oh, we_ref[...], preferred_element_type=F32)
        out_ref[...] = jnp.dot(chem, wl_ref[...], preferred_element_type=F32)

    return pl.pallas_call(
        body,
        grid=grid,
        in_specs=[
            pl.BlockSpec((1, 1, BN), lambda i: (i, 0, 0)),
            pl.BlockSpec((NT, 8), lambda i: (0, 0)),
            pl.BlockSpec((8, N_CH), lambda i: (0, 0)),
        ],
        out_specs=pl.BlockSpec((BN, N_CH), lambda i: (i, 0)),
        out_shape=jax.ShapeDtypeStruct((N, N_CH), F32),
    )


def _tc_geom(E_pad):
    grid = (E_pad // BE,)

    def body(ev_ref, out_ref):
        d = ev_ref[...][:, 0:3]
        r2s = jnp.sum(d * d, axis=1, keepdims=True) + 1e-12
        r = jnp.sqrt(r2s)
        inv_r = 1.0 / r
        nvec = (lax.broadcasted_iota(I32, (1, NUM_BASIS), 1) + 1
                ).astype(F32) * (math.pi / R_MAX)
        b = jnp.sin(r * nvec) * (PREF * inv_r)
        x = r * (1.0 / R_MAX)
        x2 = x * x
        x4 = x2 * x2
        x6 = x4 * x2
        x7 = x6 * x
        x8 = x7 * x
        fc = 1.0 - 28.0 * x6 + 48.0 * x7 - 21.0 * x8
        fc = jnp.where(x < 1.0, fc, 0.0)
        basis = b * fc
        y = d * (SQRT3 * inv_r)
        pad1 = jnp.zeros((BE, 1), F32)
        pad4 = jnp.zeros((BE, 4), F32)
        out_ref[...] = jnp.concatenate([y, pad1, basis, pad4], axis=1)

    return pl.pallas_call(
        body,
        grid=grid,
        in_specs=[pl.BlockSpec((BE, 16), lambda i: (i, 0))],
        out_specs=pl.BlockSpec((BE, 16), lambda i: (i, 0)),
        out_shape=jax.ShapeDtypeStruct((E_pad, 16), F32),
    )


def _tc_messages(E_pad, first, last):
    grid = (E_pad // BE,)
    W = 16 if first else 64
    n_out = 1 if last else 4

    def body(geo_ref, g_ref, r1_ref, r2_ref, r3_ref, out_ref):
        geo = geo_ref[...]
        basis = geo[:, 4:12]
        h = _silu(jnp.dot(basis, r1_ref[...], preferred_element_type=F32))
        h = _silu(jnp.dot(h, r2_ref[...], preferred_element_type=F32))
        w = jnp.dot(h, r3_ref[...], preferred_element_type=F32)
        w0 = w[:, 0:16]
        w1 = w[:, 16:32]
        w2 = w[:, 32:48]
        w3 = w[:, 48:64]

        yx = geo[:, 0:1]
        yy = geo[:, 1:2]
        yz = geo[:, 2:3]

        g = g_ref[...]
        s_j = g[:, 0:16]
        if first:
            out_ref[0] = w0 * s_j
            t = w2 * s_j
            out_ref[1] = t * yx
            out_ref[2] = t * yy
            out_ref[3] = t * yz
        else:
            vx = g[:, 16:32]
            vy = g[:, 32:48]
            vz = g[:, 48:64]
            dot = vx * yx + vy * yy + vz * yz
            m_s = w0 * s_j + w1 * dot
            out_ref[0] = m_s
            if not last:
                t = w2 * s_j
                out_ref[1] = t * yx + w3 * vx
                out_ref[2] = t * yy + w3 * vy
                out_ref[3] = t * yz + w3 * vz

    return pl.pallas_call(
        body,
        grid=grid,
        in_specs=[
            pl.BlockSpec((BE, 16), lambda i: (i, 0)),
            pl.BlockSpec((BE, W), lambda i: (i, 0)),
            pl.BlockSpec((NUM_BASIS, 64), lambda i: (0, 0)),
            pl.BlockSpec((64, 64), lambda i: (0, 0)),
            pl.BlockSpec((64, 64), lambda i: (0, 0)),
        ],
        out_specs=pl.BlockSpec((n_out, BE, 16), lambda i: (0, i, 0)),
        out_shape=jax.ShapeDtypeStruct((n_out, E_pad, 16), F32),
    )


def _tc_node_first(N, NP):
    grid = (N // BN,)

    def body(agg_ref, s_ref, wss_ref, wscs_ref, wsv_ref, out_ref):
        a = agg_ref[...]
        s = s_ref[...]
        pre_s = (jnp.dot(a[0], wss_ref[...], preferred_element_type=F32)
                 * INV_AVG
                 + jnp.dot(s, wscs_ref[...], preferred_element_type=F32))
        sg = jax.nn.sigmoid(pre_s)
        s1 = pre_s * sg
        wsv = wsv_ref[...]
        sgi = sg * INV_AVG
        vx = jnp.dot(a[1], wsv, preferred_element_type=F32) * sgi
        vy = jnp.dot(a[2], wsv, preferred_element_type=F32) * sgi
        vz = jnp.dot(a[3], wsv, preferred_element_type=F32) * sgi
        out_ref[...] = jnp.concatenate([s1, vx, vy, vz], axis=1)

    return pl.pallas_call(
        body,
        grid=grid,
        in_specs=[
            pl.BlockSpec((4, BN, 16), lambda i: (0, i, 0)),
            pl.BlockSpec((BN, 16), lambda i: (i, 0)),
            pl.BlockSpec((16, 16), lambda i: (0, 0)),
            pl.BlockSpec((16, 16), lambda i: (0, 0)),
            pl.BlockSpec((16, 16), lambda i: (0, 0)),
        ],
        out_specs=pl.BlockSpec((BN, 64), lambda i: (i, 0)),
        out_shape=jax.ShapeDtypeStruct((N, 64), F32),
    )


def _tc_node_mid(N, NP):
    grid = (N // BN,)

    def body(agg_ref, st_ref, wss_ref, wscs_ref, wsv_ref, wscv_ref, out_ref):
        a = agg_ref[...]
        st = st_ref[...]
        s = st[:, 0:16]
        pre_s = (jnp.dot(a[0], wss_ref[...], preferred_element_type=F32)
                 * INV_AVG
                 + jnp.dot(s, wscs_ref[...], preferred_element_type=F32))
        sg = jax.nn.sigmoid(pre_s)
        s1 = pre_s * sg
        wsv = wsv_ref[...]
        wscv = wscv_ref[...]
        vx = (jnp.dot(a[1], wsv, preferred_element_type=F32) * INV_AVG
              + jnp.dot(st[:, 16:32], wscv, preferred_element_type=F32)) * sg
        vy = (jnp.dot(a[2], wsv, preferred_element_type=F32) * INV_AVG
              + jnp.dot(st[:, 32:48], wscv, preferred_element_type=F32)) * sg
        vz = (jnp.dot(a[3], wsv, preferred_element_type=F32) * INV_AVG
              + jnp.dot(st[:, 48:64], wscv, preferred_element_type=F32)) * sg
        out_ref[...] = jnp.concatenate([s1, vx, vy, vz], axis=1)

    return pl.pallas_call(
        body,
        grid=grid,
        in_specs=[
            pl.BlockSpec((4, BN, 16), lambda i: (0, i, 0)),
            pl.BlockSpec((BN, 64), lambda i: (i, 0)),
            pl.BlockSpec((16, 16), lambda i: (0, 0)),
            pl.BlockSpec((16, 16), lambda i: (0, 0)),
            pl.BlockSpec((16, 16), lambda i: (0, 0)),
            pl.BlockSpec((16, 16), lambda i: (0, 0)),
        ],
        out_specs=pl.BlockSpec((BN, 64), lambda i: (i, 0)),
        out_shape=jax.ShapeDtypeStruct((N, 64), F32),
    )


def _tc_node_last(N, NP, NG):
    grid = (N // BN,)

    def body(agg_ref, st_ref, wss_ref, wscs_ref, w1_ref, w2_ref, b_ref,
             out_ref):
        a = agg_ref[...]
        agg_s = a[0] + a[1]
        s = st_ref[...][:, 0:16]
        pre_s = (jnp.dot(agg_s, wss_ref[...], preferred_element_type=F32)
                 * INV_AVG
                 + jnp.dot(s, wscs_ref[...], preferred_element_type=F32))
        s2 = pre_s * jax.nn.sigmoid(pre_s)
        h = jnp.dot(s2, w1_ref[...], preferred_element_type=F32)
        pa = jnp.dot(h, w2_ref[...], preferred_element_type=F32)
        bid = b_ref[0, 0]
        oh = (bid[:, None] == lax.broadcasted_iota(I32, (BN, NG), 1)).astype(F32)
        e = jnp.sum(oh * pa, axis=0)

        @pl.when(pl.program_id(0) == 0)
        def _():
            out_ref[...] = jnp.zeros_like(out_ref)

        out_ref[...] += e[None, :]

    return pl.pallas_call(
        body,
        grid=grid,
        in_specs=[
            pl.BlockSpec((2, BN, 16), lambda i: (0, i, 0)),
            pl.BlockSpec((BN, 64), lambda i: (i, 0)),
            pl.BlockSpec((16, 16), lambda i: (0, 0)),
            pl.BlockSpec((16, 16), lambda i: (0, 0)),
            pl.BlockSpec((16, 8), lambda i: (0, 0)),
            pl.BlockSpec((8, 1), lambda i: (0, 0)),
            pl.BlockSpec((1, 1, BN), lambda i: (i, 0, 0)),
        ],
        out_specs=pl.BlockSpec((1, NG), lambda i: (0, 0)),
        out_shape=jax.ShapeDtypeStruct((1, NG), F32),
    )


def kernel(pos, atomic_numbers, batch, edge_index, W_embed, W_lift,
           R1, R2, R3, W_self_s, W_self_v, W_sc_s, W_sc_v, W_out1, W_out2):
    N = pos.shape[0]
    E = edge_index.shape[1]
    NT = W_embed.shape[0]
    NG = 64

    E_pad = _ru(E, NW * CHUNK * 2)
    NP = _ru(N + 1, NSUB * CHUNK)

    src = jnp.concatenate(
        [edge_index[0].astype(I32), jnp.zeros((E_pad - E,), I32)])
    dst = jnp.concatenate(
        [edge_index[1].astype(I32), jnp.full((E_pad - E,), N, I32)])
    pos4 = jnp.zeros((N + 8, 16), F32).at[:N, :3].set(pos.astype(F32))
    atom3 = atomic_numbers.astype(I32).reshape(N // BN, 1, BN)
    batch3 = batch.astype(I32).reshape(N // BN, 1, BN)

    s0 = _tc_embed(N, NT)(atom3, W_embed, W_lift)
    ev = _sc_edge_vec(E_pad, N + 8)(src, dst, pos4)
    geo = _tc_geom(E_pad)(ev)

    g0 = _sc_gather(E_pad, 16)(src, s0)
    m0 = _tc_messages(E_pad, first=True, last=False)(geo, g0, R1[0], R2[0], R3[0])
    agg0 = _sc_scatter(E_pad, NP, "pair")(dst, m0)
    st1 = _tc_node_first(N, NP)(
        agg0.reshape(4, NP, 16), s0, W_self_s[0], W_sc_s[0], W_self_v[0])

    g1 = _sc_gather(E_pad, 64)(src, st1)
    m1 = _tc_messages(E_pad, first=False, last=False)(geo, g1, R1[1], R2[1], R3[1])
    agg1 = _sc_scatter(E_pad, NP, "pair")(dst, m1)
    st2 = _tc_node_mid(N, NP)(
        agg1.reshape(4, NP, 16), st1, W_self_s[1], W_sc_s[1], W_self_v[1],
        W_sc_v[1])

    g2 = _sc_gather(E_pad, 64)(src, st2)
    m2 = _tc_messages(E_pad, first=False, last=True)(geo, g2, R1[2], R2[2], R3[2])
    agg2 = _sc_scatter(E_pad, NP, "half")(dst, m2)
    energy = _tc_node_last(N, NP, NG)(
        agg2.reshape(2, NP, 16), st2, W_self_s[2], W_sc_s[2], W_out1, W_out2,
        batch3)

    return energy.reshape(NG)


# speedup vs baseline: 22.0513x; 1.0952x over previous
import functools
import math

import jax
import jax.numpy as jnp
from jax import lax
from jax.experimental import pallas as pl
from jax.experimental.pallas import tpu as pltpu
from jax.experimental.pallas import tpu_sc as plsc

F32 = jnp.float32
I32 = jnp.int32

NCORES = 2
NSUB = 16
NW = NCORES * NSUB
LANES = 16
CHUNK = 128

R_MAX = 4.0
NUM_BASIS = 8
P_CUT = 6.0
N_CH = 16
INV_AVG = 1.0 / 16.0
SQRT3 = math.sqrt(3.0)
PREF = math.sqrt(2.0 / R_MAX)

BE = 4096
BN = 2000


def _ru(x, m):
    return (x + m - 1) // m * m


def _mesh():
    return plsc.VectorSubcoreMesh(
        core_axis_name="c", subcore_axis_name="s",
        num_cores=NCORES, num_subcores=NSUB)


_SC_PARAMS = pltpu.CompilerParams(use_tc_tiling_on_sc=False)


def _silu(x):
    return x * jax.nn.sigmoid(x)


def _sc_edge_vec(E_pad, NPOS):
    EW = E_pad // NW
    NCH = EW // CHUNK

    scratch = (
        [pltpu.VMEM((CHUNK,), I32) for _ in range(4)]
        + [pltpu.VMEM((CHUNK, 16), F32) for _ in range(6)]
        + [pltpu.SemaphoreType.DMA] * 6
    )

    @functools.partial(
        pl.kernel,
        out_type=jax.ShapeDtypeStruct((E_pad, 16), F32),
        mesh=_mesh(),
        compiler_params=_SC_PARAMS,
        scratch_types=scratch,
    )
    def k(src_h, dst_h, pos_h, ev_h,
          sx0, sx1, dx0, dx1, ps0, ps1, pd0, pd1, db0, db1,
          si0, si1, sg0, sg1, so0, so1):
        sidx = [sx0, sx1]
        didx = [dx0, dx1]
        ps = [ps0, ps1]
        pd = [pd0, pd1]
        db = [db0, db1]
        semi = [si0, si1]
        semg = [sg0, sg1]
        semo = [so0, so1]

        cid = lax.axis_index("c")
        sid = lax.axis_index("s")
        wid = sid * NCORES + cid
        base = wid * EW

        def issue_idx(b, c):
            off = base + c * CHUNK
            pltpu.async_copy(src_h.at[pl.ds(off, CHUNK)], sidx[b], semi[b])
            pltpu.async_copy(dst_h.at[pl.ds(off, CHUNK)], didx[b], semi[b])

        def wait_idx(b):
            pltpu.make_async_copy(src_h.at[pl.ds(0, CHUNK)], sidx[b], semi[b]).wait()
            pltpu.make_async_copy(dst_h.at[pl.ds(0, CHUNK)], didx[b], semi[b]).wait()

        def drain_out(b):
            pltpu.make_async_copy(db[b], ev_h.at[pl.ds(0, CHUNK), :], semo[b]).wait()

        for b in range(2):
            issue_idx(b, b)

        @pl.loop(0, NCH, step=2)
        def _(g):
            for b in range(2):
                c = g + b
                wait_idx(b)

                pltpu.async_copy(pos_h.at[sidx[b]], ps[b], semg[b])
                pltpu.async_copy(pos_h.at[didx[b]], pd[b], semg[b])
                pltpu.make_async_copy(pos_h.at[sidx[b]], ps[b], semg[b]).wait()
                pltpu.make_async_copy(pos_h.at[didx[b]], pd[b], semg[b]).wait()

                @pl.when(c >= 2)
                def _():
                    drain_out(b)

                for i in range(CHUNK):
                    db[b][i, :] = ps[b][i, :] - pd[b][i, :]
                pltpu.async_copy(db[b],
                                 ev_h.at[pl.ds(base + c * CHUNK, CHUNK), :],
                                 semo[b])

                @pl.when(c + 2 < NCH)
                def _():
                    issue_idx(b, c + 2)

        for b in range(2):
            drain_out(b)

    return k


def _sc_gather(E_pad, W):
    EW = E_pad // NW
    NCH = EW // CHUNK

    scratch = (
        [pltpu.VMEM((CHUNK,), I32) for _ in range(2)]
        + [pltpu.VMEM((CHUNK, W), F32) for _ in range(2)]
        + [pltpu.SemaphoreType.DMA] * 6
    )

    @functools.partial(
        pl.kernel,
        out_type=jax.ShapeDtypeStruct((E_pad, W), F32),
        mesh=_mesh(),
        compiler_params=_SC_PARAMS,
        scratch_types=scratch,
    )
    def k(src_h, tab_h, out_h,
          sx0, sx1, gb0, gb1, si0, si1, sg0, sg1, so0, so1):
        sidx = [sx0, sx1]
        gb = [gb0, gb1]
        semi = [si0, si1]
        semg = [sg0, sg1]
        semo = [so0, so1]

        cid = lax.axis_index("c")
        sid = lax.axis_index("s")
        wid = sid * NCORES + cid
        base = wid * EW

        def issue_idx(b, c):
            pltpu.async_copy(src_h.at[pl.ds(base + c * CHUNK, CHUNK)],
                             sidx[b], semi[b])

        def wait_idx(b):
            pltpu.make_async_copy(src_h.at[pl.ds(0, CHUNK)], sidx[b], semi[b]).wait()

        for b in range(2):
            issue_idx(b, b)

        @pl.loop(0, NCH, step=2)
        def _(g):
            for b in range(2):
                c = g + b
                wait_idx(b)

                @pl.when(c >= 2)
                def _():
                    pltpu.make_async_copy(
                        gb[b], out_h.at[pl.ds(0, CHUNK), :], semo[b]).wait()

                pltpu.async_copy(tab_h.at[sidx[b]], gb[b], semg[b])
                pltpu.make_async_copy(tab_h.at[sidx[b]], gb[b], semg[b]).wait()
                pltpu.async_copy(gb[b],
                                 out_h.at[pl.ds(base + c * CHUNK, CHUNK), :],
                                 semo[b])

                @pl.when(c + 2 < NCH)
                def _():
                    issue_idx(b, c + 2)

        for b in range(2):
            pltpu.make_async_copy(gb[b], out_h.at[pl.ds(0, CHUNK), :], semo[b]).wait()

    return k


def _sc_scatter(E_pad, NP, mode):
    SR = NP // NSUB
    SRC = SR // CHUNK

    if mode == "pair":
        ET = E_pad // NSUB
        n_planes = 4
        n_comp = 4
    else:
        ET = E_pad // NW
        n_planes = 2
        n_comp = 1
    NCH = ET // CHUNK

    MW = 16 * n_comp
    scratch = (
        [pltpu.VMEM((CHUNK,), I32) for _ in range(2)]
        + [pltpu.VMEM((CHUNK, MW), F32) for _ in range(2)]
        + [pltpu.VMEM((CHUNK, 16), F32)]
        + [pltpu.VMEM((CHUNK, 16), F32)]
        + [pltpu.VMEM_SHARED((NP, 16), F32)]
        + [pltpu.SemaphoreType.DMA] * 2
    )

    @functools.partial(
        pl.kernel,
        out_type=jax.ShapeDtypeStruct((n_planes * NP, 16), F32),
        mesh=_mesh(),
        compiler_params=_SC_PARAMS,
        scratch_types=scratch,
    )
    def k(dst_h, m_h, agg_h, dx0, dx1, mb0, mb1, rp, zb, acc, sl0, sl1):
        didx = [dx0, dx1]
        mb = [mb0, mb1]
        seml = [sl0, sl1]

        cid = lax.axis_index("c")
        sid = lax.axis_index("s")

        for i in range(CHUNK):
            zb[i, :] = jnp.zeros((LANES,), F32)

        n_comp_loc = 2 if mode == "pair" else 1
        for t in range(n_comp_loc):
            if mode == "pair":
                comp = cid * 2 + t
                ebase = sid * ET
                plane = comp
            else:
                comp = 0
                ebase = (sid * NCORES + cid) * ET
                plane = cid

            for kk in range(SRC):
                pltpu.sync_copy(zb, acc.at[pl.ds(sid * SR + kk * CHUNK, CHUNK), :])
            plsc.subcore_barrier()

            def issue_loads(b, c):
                pltpu.async_copy(dst_h.at[pl.ds(ebase + c * CHUNK, CHUNK)],
                                 didx[b], seml[b])
                pltpu.async_copy(m_h.at[pl.ds(ebase + c * CHUNK, CHUNK), :],
                                 mb[b], seml[b])

            def wait_loads(b):
                pltpu.make_async_copy(dst_h.at[pl.ds(0, CHUNK)], didx[b],
                                      seml[b]).wait()
                pltpu.make_async_copy(m_h.at[pl.ds(0, CHUNK), :], mb[b],
                                      seml[b]).wait()

            def repack(b, col):
                for j in range(CHUNK):
                    rp[j, :] = mb[b][j, col:col + 16]

            for b in range(2):
                issue_loads(b, b)

            @pl.loop(0, NCH, step=2)
            def _(g):
                for b in range(2):
                    c = g + b
                    wait_loads(b)
                    if mode == "pair":
                        @pl.when(cid == 0)
                        def _():
                            repack(b, t * 16)

                        @pl.when(cid == 1)
                        def _():
                            repack(b, 32 + t * 16)

                        pltpu.sync_copy(rp, acc.at[didx[b]], add=True)
                    else:
                        pltpu.sync_copy(mb[b], acc.at[didx[b]], add=True)

                    @pl.when(c + 2 < NCH)
                    def _():
                        issue_loads(b, c + 2)

            plsc.subcore_barrier()
            pltpu.sync_copy(
                acc.at[pl.ds(sid * SR, SR), :],
                agg_h.at[pl.ds(plane * NP + sid * SR, SR), :])
            plsc.subcore_barrier()

    return k


def _tc_embed(N, NT):
    grid = (N // BN,)

    def body(a_ref, we_ref, wl_ref, out_ref):
        an = a_ref[0, 0]
        oh = (an[:, None] == lax.broadcasted_iota(I32, (BN, NT), 1)).astype(F32)
        chem = jnp.dot(oh, we_ref[...], preferred_element_type=F32)
        out_ref[...] = jnp.dot(chem, wl_ref[...], preferred_element_type=F32)

    return pl.pallas_call(
        body,
        grid=grid,
        in_specs=[
            pl.BlockSpec((1, 1, BN), lambda i: (i, 0, 0)),
            pl.BlockSpec((NT, 8), lambda i: (0, 0)),
            pl.BlockSpec((8, N_CH), lambda i: (0, 0)),
        ],
        out_specs=pl.BlockSpec((BN, N_CH), lambda i: (i, 0)),
        out_shape=jax.ShapeDtypeStruct((N, N_CH), F32),
    )


def _tc_geom(E_pad):
    grid = (E_pad // BE,)

    def body(ev_ref, out_ref):
        d = ev_ref[...][:, 0:3]
        r2s = jnp.sum(d * d, axis=1, keepdims=True) + 1e-12
        r = jnp.sqrt(r2s)
        inv_r = 1.0 / r
        nvec = (lax.broadcasted_iota(I32, (1, NUM_BASIS), 1) + 1
                ).astype(F32) * (math.pi / R_MAX)
        b = jnp.sin(r * nvec) * (PREF * inv_r)
        x = r * (1.0 / R_MAX)
        x2 = x * x
        x4 = x2 * x2
        x6 = x4 * x2
        x7 = x6 * x
        x8 = x7 * x
        fc = 1.0 - 28.0 * x6 + 48.0 * x7 - 21.0 * x8
        fc = jnp.where(x < 1.0, fc, 0.0)
        basis = b * fc
        y = d * (SQRT3 * inv_r)
        pad1 = jnp.zeros((BE, 1), F32)
        pad4 = jnp.zeros((BE, 4), F32)
        out_ref[...] = jnp.concatenate([y, pad1, basis, pad4], axis=1)

    return pl.pallas_call(
        body,
        grid=grid,
        in_specs=[pl.BlockSpec((BE, 16), lambda i: (i, 0))],
        out_specs=pl.BlockSpec((BE, 16), lambda i: (i, 0)),
        out_shape=jax.ShapeDtypeStruct((E_pad, 16), F32),
    )


def _tc_messages(E_pad, first, last):
    grid = (E_pad // BE,)
    W = 16 if first else 64
    n_out = 1 if last else 4

    def body(geo_ref, g_ref, r1_ref, r2_ref, r3_ref, out_ref):
        geo = geo_ref[...]
        basis = geo[:, 4:12]
        h = _silu(jnp.dot(basis, r1_ref[...], preferred_element_type=F32))
        h = _silu(jnp.dot(h, r2_ref[...], preferred_element_type=F32))
        w = jnp.dot(h, r3_ref[...], preferred_element_type=F32)
        w0 = w[:, 0:16]
        w1 = w[:, 16:32]
        w2 = w[:, 32:48]
        w3 = w[:, 48:64]

        yx = geo[:, 0:1]
        yy = geo[:, 1:2]
        yz = geo[:, 2:3]

        g = g_ref[...]
        s_j = g[:, 0:16]
        if first:
            t = w2 * s_j
            out_ref[...] = jnp.concatenate(
                [w0 * s_j, t * yx, t * yy, t * yz], axis=1)
        else:
            vx = g[:, 16:32]
            vy = g[:, 32:48]
            vz = g[:, 48:64]
            dot = vx * yx + vy * yy + vz * yz
            m_s = w0 * s_j + w1 * dot
            if last:
                out_ref[...] = m_s
            else:
                t = w2 * s_j
                out_ref[...] = jnp.concatenate(
                    [m_s, t * yx + w3 * vx, t * yy + w3 * vy,
                     t * yz + w3 * vz], axis=1)

    return pl.pallas_call(
        body,
        grid=grid,
        in_specs=[
            pl.BlockSpec((BE, 16), lambda i: (i, 0)),
            pl.BlockSpec((BE, W), lambda i: (i, 0)),
            pl.BlockSpec((NUM_BASIS, 64), lambda i: (0, 0)),
            pl.BlockSpec((64, 64), lambda i: (0, 0)),
            pl.BlockSpec((64, 64), lambda i: (0, 0)),
        ],
        out_specs=pl.BlockSpec((BE, 16 * n_out), lambda i: (i, 0)),
        out_shape=jax.ShapeDtypeStruct((E_pad, 16 * n_out), F32),
    )


def _tc_node_first(N, NP):
    grid = (N // BN,)

    def body(agg_ref, s_ref, wss_ref, wscs_ref, wsv_ref, out_ref):
        a = agg_ref[...]
        s = s_ref[...]
        pre_s = (jnp.dot(a[0], wss_ref[...], preferred_element_type=F32)
                 * INV_AVG
                 + jnp.dot(s, wscs_ref[...], preferred_element_type=F32))
        sg = jax.nn.sigmoid(pre_s)
        s1 = pre_s * sg
        wsv = wsv_ref[...]
        sgi = sg * INV_AVG
        vx = jnp.dot(a[1], wsv, preferred_element_type=F32) * sgi
        vy = jnp.dot(a[2], wsv, preferred_element_type=F32) * sgi
        vz = jnp.dot(a[3], wsv, preferred_element_type=F32) * sgi
        out_ref[...] = jnp.concatenate([s1, vx, vy, vz], axis=1)

    return pl.pallas_call(
        body,
        grid=grid,
        in_specs=[
            pl.BlockSpec((4, BN, 16), lambda i: (0, i, 0)),
            pl.BlockSpec((BN, 16), lambda i: (i, 0)),
            pl.BlockSpec((16, 16), lambda i: (0, 0)),
            pl.BlockSpec((16, 16), lambda i: (0, 0)),
            pl.BlockSpec((16, 16), lambda i: (0, 0)),
        ],
        out_specs=pl.BlockSpec((BN, 64), lambda i: (i, 0)),
        out_shape=jax.ShapeDtypeStruct((N, 64), F32),
    )


def _tc_node_mid(N, NP):
    grid = (N // BN,)

    def body(agg_ref, st_ref, wss_ref, wscs_ref, wsv_ref, wscv_ref, out_ref):
        a = agg_ref[...]
        st = st_ref[...]
        s = st[:, 0:16]
        pre_s = (jnp.dot(a[0], wss_ref[...], preferred_element_type=F32)
                 * INV_AVG
                 + jnp.dot(s, wscs_ref[...], preferred_element_type=F32))
        sg = jax.nn.sigmoid(pre_s)
        s1 = pre_s * sg
        wsv = wsv_ref[...]
        wscv = wscv_ref[...]
        vx = (jnp.dot(a[1], wsv, preferred_element_type=F32) * INV_AVG
              + jnp.dot(st[:, 16:32], wscv, preferred_element_type=F32)) * sg
        vy = (jnp.dot(a[2], wsv, preferred_element_type=F32) * INV_AVG
              + jnp.dot(st[:, 32:48], wscv, preferred_element_type=F32)) * sg
        vz = (jnp.dot(a[3], wsv, preferred_element_type=F32) * INV_AVG
              + jnp.dot(st[:, 48:64], wscv, preferred_element_type=F32)) * sg
        out_ref[...] = jnp.concatenate([s1, vx, vy, vz], axis=1)

    return pl.pallas_call(
        body,
        grid=grid,
        in_specs=[
            pl.BlockSpec((4, BN, 16), lambda i: (0, i, 0)),
            pl.BlockSpec((BN, 64), lambda i: (i, 0)),
            pl.BlockSpec((16, 16), lambda i: (0, 0)),
            pl.BlockSpec((16, 16), lambda i: (0, 0)),
            pl.BlockSpec((16, 16), lambda i: (0, 0)),
            pl.BlockSpec((16, 16), lambda i: (0, 0)),
        ],
        out_specs=pl.BlockSpec((BN, 64), lambda i: (i, 0)),
        out_shape=jax.ShapeDtypeStruct((N, 64), F32),
    )


def _tc_node_last(N, NP, NG):
    grid = (N // BN,)

    def body(agg_ref, st_ref, wss_ref, wscs_ref, w1_ref, w2_ref, b_ref,
             out_ref):
        a = agg_ref[...]
        agg_s = a[0] + a[1]
        s = st_ref[...][:, 0:16]
        pre_s = (jnp.dot(agg_s, wss_ref[...], preferred_element_type=F32)
                 * INV_AVG
                 + jnp.dot(s, wscs_ref[...], preferred_element_type=F32))
        s2 = pre_s * jax.nn.sigmoid(pre_s)
        h = jnp.dot(s2, w1_ref[...], preferred_element_type=F32)
        pa = jnp.dot(h, w2_ref[...], preferred_element_type=F32)
        bid = b_ref[0, 0]
        oh = (bid[:, None] == lax.broadcasted_iota(I32, (BN, NG), 1)).astype(F32)
        e = jnp.sum(oh * pa, axis=0)

        @pl.when(pl.program_id(0) == 0)
        def _():
            out_ref[...] = jnp.zeros_like(out_ref)

        out_ref[...] += e[None, :]

    return pl.pallas_call(
        body,
        grid=grid,
        in_specs=[
            pl.BlockSpec((2, BN, 16), lambda i: (0, i, 0)),
            pl.BlockSpec((BN, 64), lambda i: (i, 0)),
            pl.BlockSpec((16, 16), lambda i: (0, 0)),
            pl.BlockSpec((16, 16), lambda i: (0, 0)),
            pl.BlockSpec((16, 8), lambda i: (0, 0)),
            pl.BlockSpec((8, 1), lambda i: (0, 0)),
            pl.BlockSpec((1, 1, BN), lambda i: (i, 0, 0)),
        ],
        out_specs=pl.BlockSpec((1, NG), lambda i: (0, 0)),
        out_shape=jax.ShapeDtypeStruct((1, NG), F32),
    )


def kernel(pos, atomic_numbers, batch, edge_index, W_embed, W_lift,
           R1, R2, R3, W_self_s, W_self_v, W_sc_s, W_sc_v, W_out1, W_out2):
    N = pos.shape[0]
    E = edge_index.shape[1]
    NT = W_embed.shape[0]
    NG = 64

    E_pad = _ru(E, NW * CHUNK * 2)
    NP = _ru(N + 1, NSUB * CHUNK)

    src = jnp.concatenate(
        [edge_index[0].astype(I32), jnp.zeros((E_pad - E,), I32)])
    dst = jnp.concatenate(
        [edge_index[1].astype(I32), jnp.full((E_pad - E,), N, I32)])
    pos4 = jnp.zeros((N + 8, 16), F32).at[:N, :3].set(pos.astype(F32))
    atom3 = atomic_numbers.astype(I32).reshape(N // BN, 1, BN)
    batch3 = batch.astype(I32).reshape(N // BN, 1, BN)

    s0 = _tc_embed(N, NT)(atom3, W_embed, W_lift)
    ev = _sc_edge_vec(E_pad, N + 8)(src, dst, pos4)
    geo = _tc_geom(E_pad)(ev)

    g0 = _sc_gather(E_pad, 16)(src, s0)
    m0 = _tc_messages(E_pad, first=True, last=False)(geo, g0, R1[0], R2[0], R3[0])
    agg0 = _sc_scatter(E_pad, NP, "pair")(dst, m0)
    st1 = _tc_node_first(N, NP)(
        agg0.reshape(4, NP, 16), s0, W_self_s[0], W_sc_s[0], W_self_v[0])

    g1 = _sc_gather(E_pad, 64)(src, st1)
    m1 = _tc_messages(E_pad, first=False, last=False)(geo, g1, R1[1], R2[1], R3[1])
    agg1 = _sc_scatter(E_pad, NP, "pair")(dst, m1)
    st2 = _tc_node_mid(N, NP)(
        agg1.reshape(4, NP, 16), st1, W_self_s[1], W_sc_s[1], W_self_v[1],
        W_sc_v[1])

    g2 = _sc_gather(E_pad, 64)(src, st2)
    m2 = _tc_messages(E_pad, first=False, last=True)(geo, g2, R1[2], R2[2], R3[2])
    agg2 = _sc_scatter(E_pad, NP, "half")(dst, m2)
    energy = _tc_node_last(N, NP, NG)(
        agg2.reshape(2, NP, 16), st2, W_self_s[2], W_sc_s[2], W_out1, W_out2,
        batch3)

    return energy.reshape(NG)
